# Initial kernel scaffold; baseline (speedup 1.0000x reference)
import jax
import jax.numpy as jnp
from jax.experimental import pallas as pl


def kernel(X, node_idx, edge_idx, W, b):
    raise NotImplementedError("write your pallas kernel here")


# speedup vs baseline: 25.1368x; 25.1368x over previous
import functools

import jax
import jax.numpy as jnp
from jax import lax
from jax.experimental import pallas as pl
from jax.experimental.pallas import tpu as pltpu
from jax.experimental.pallas import tpu_sc as plsc

N_NODES = 10000
N_PAIRS = 160000
N_HEDGES = 20000
D = 256

LANES = 16
NWORK = 32
PPW = N_PAIRS // NWORK
NCHUNK = PPW // LANES
TAIL = PPW - NCHUNK * LANES
PPW_PAD = (NCHUNK + 1) * LANES
H_PAD = 20480


def _matvec_body(x_ref, wt_ref, y_ref):
    y_ref[...] = jnp.dot(x_ref[...], wt_ref[...],
                         preferred_element_type=jnp.float32)


def _sc_body(y_hbm, nidx_hbm, eidx_hbm, sums_hbm, cnts_hbm,
             y_v, nidx_v, eidx_v, acc_v, cnt_v):
    wid = lax.axis_index("s") * 2 + lax.axis_index("c")
    base = wid * PPW

    pltpu.sync_copy(y_hbm, y_v.at[pl.ds(0, N_NODES)])
    pltpu.sync_copy(nidx_hbm.at[pl.ds(base, PPW)], nidx_v.at[pl.ds(0, PPW)])
    pltpu.sync_copy(eidx_hbm.at[pl.ds(base, PPW)], eidx_v.at[pl.ds(0, PPW)])

    zeros = jnp.zeros((LANES,), jnp.float32)
    ones = jnp.ones((LANES,), jnp.float32)

    def zero_body(i, carry):
        acc_v[pl.ds(i * LANES, LANES)] = zeros
        cnt_v[pl.ds(i * LANES, LANES)] = zeros
        return carry

    lax.fori_loop(0, H_PAD // LANES, zero_body, 0)

    def pair_body(i, carry):
        off = i * LANES
        ni = nidx_v[pl.ds(off, LANES)]
        ei = eidx_v[pl.ds(off, LANES)]
        vals = plsc.load_gather(y_v, [ni])
        plsc.addupdate_scatter(acc_v, [ei], vals)
        plsc.addupdate_scatter(cnt_v, [ei], ones)
        return carry

    lax.fori_loop(0, NCHUNK, pair_body, 0)

    mask = lax.iota(jnp.int32, LANES) < TAIL
    off = NCHUNK * LANES
    ni = jnp.where(mask, nidx_v[pl.ds(off, LANES)], 0)
    ei = jnp.where(mask, eidx_v[pl.ds(off, LANES)], 0)
    vals = plsc.load_gather(y_v, [ni])
    plsc.addupdate_scatter(acc_v, [ei], vals, mask=mask)
    plsc.addupdate_scatter(cnt_v, [ei], ones, mask=mask)

    pltpu.sync_copy(acc_v, sums_hbm.at[wid])
    pltpu.sync_copy(cnt_v, cnts_hbm.at[wid])


def _finalize_body(s_ref, c_ref, b_ref, out_ref):
    s = jnp.sum(s_ref[...], axis=0)
    c = jnp.sum(c_ref[...], axis=0)
    z = s / jnp.maximum(c, 1.0) + b_ref[0, 0]
    out_ref[...] = jax.nn.sigmoid(z)


@jax.jit
def kernel(X, node_idx, edge_idx, W, b):
    y2d = pl.pallas_call(
        _matvec_body,
        grid=(5,),
        in_specs=[
            pl.BlockSpec((2000, D), lambda i: (i, 0)),
            pl.BlockSpec((D, 1), lambda i: (0, 0)),
        ],
        out_specs=pl.BlockSpec((2000, 1), lambda i: (i, 0)),
        out_shape=jax.ShapeDtypeStruct((N_NODES, 1), jnp.float32),
    )(X, W.reshape(D, 1))
    y = y2d.reshape(N_NODES)

    mesh = plsc.VectorSubcoreMesh(core_axis_name="c", subcore_axis_name="s")
    sc_seg = pl.kernel(
        _sc_body,
        out_type=(
            jax.ShapeDtypeStruct((NWORK, H_PAD), jnp.float32),
            jax.ShapeDtypeStruct((NWORK, H_PAD), jnp.float32),
        ),
        mesh=mesh,
        compiler_params=pltpu.CompilerParams(needs_layout_passes=False),
        scratch_types=[
            pltpu.VMEM((10240,), jnp.float32),
            pltpu.VMEM((PPW_PAD,), jnp.int32),
            pltpu.VMEM((PPW_PAD,), jnp.int32),
            pltpu.VMEM((H_PAD,), jnp.float32),
            pltpu.VMEM((H_PAD,), jnp.float32),
        ],
    )
    sums, cnts = sc_seg(y, node_idx, edge_idx)

    out = pl.pallas_call(
        _finalize_body,
        in_specs=[
            pl.BlockSpec((NWORK, 160, 128), lambda: (0, 0, 0)),
            pl.BlockSpec((NWORK, 160, 128), lambda: (0, 0, 0)),
            pl.BlockSpec(memory_space=pltpu.SMEM),
        ],
        out_specs=pl.BlockSpec((160, 128), lambda: (0, 0)),
        out_shape=jax.ShapeDtypeStruct((160, 128), jnp.float32),
    )(sums.reshape(NWORK, 160, 128), cnts.reshape(NWORK, 160, 128),
      b.reshape(1, 1))

    return out.reshape(H_PAD)[:N_HEDGES].reshape(N_HEDGES, 1)


# speedup vs baseline: 29.2319x; 1.1629x over previous
import functools

import jax
import jax.numpy as jnp
from jax import lax
from jax.experimental import pallas as pl
from jax.experimental.pallas import tpu as pltpu
from jax.experimental.pallas import tpu_sc as plsc

N_NODES = 10000
N_PAIRS = 160000
N_HEDGES = 20000
D = 256

LANES = 16
NWORK = 32
PPW = N_PAIRS // NWORK
NCHUNK = PPW // LANES
TAIL = PPW - NCHUNK * LANES
PPW_PAD = (NCHUNK + 1) * LANES
H_ROWS = 160
H_COLS = 128
H_PAD = H_ROWS * H_COLS


def _matvec_body(x_ref, wt_ref, y_ref):
    y_ref[...] = jnp.dot(x_ref[...], wt_ref[...],
                         preferred_element_type=jnp.float32)


def _sc_body(y_hbm, nidx_hbm, eidx_hbm, sums_hbm, cnts_hbm,
             y_v, nidx_v, eidx_v, acc_v, cnt_v):
    wid = lax.axis_index("s") * 2 + lax.axis_index("c")
    base = wid * PPW

    pltpu.sync_copy(y_hbm, y_v.at[pl.ds(0, N_NODES)])
    pltpu.sync_copy(nidx_hbm.at[pl.ds(base, PPW)], nidx_v.at[pl.ds(0, PPW)])
    pltpu.sync_copy(eidx_hbm.at[pl.ds(base, PPW)], eidx_v.at[pl.ds(0, PPW)])

    zeros = jnp.zeros((LANES,), jnp.float32)
    ones = jnp.ones((LANES,), jnp.float32)

    def zero_body(r, carry):
        for c in range(H_COLS // LANES):
            acc_v[r, pl.ds(c * LANES, LANES)] = zeros
            cnt_v[r, pl.ds(c * LANES, LANES)] = zeros
        return carry

    lax.fori_loop(0, H_ROWS, zero_body, 0)

    def do_chunk(off):
        ni = nidx_v[pl.ds(off, LANES)]
        ei = eidx_v[pl.ds(off, LANES)]
        er = lax.shift_right_logical(ei, 7)
        ec = lax.bitwise_and(ei, 127)
        vals = plsc.load_gather(y_v, [ni])
        plsc.addupdate_scatter(acc_v, [er, ec], vals)
        plsc.addupdate_scatter(cnt_v, [er, ec], ones)

    UNROLL = 4

    def pair_body(i, carry):
        for u in range(UNROLL):
            do_chunk(i * (LANES * UNROLL) + u * LANES)
        return carry

    lax.fori_loop(0, NCHUNK // UNROLL, pair_body, 0)

    mask = lax.iota(jnp.int32, LANES) < TAIL
    off = NCHUNK * LANES
    ni = jnp.where(mask, nidx_v[pl.ds(off, LANES)], 0)
    ei = jnp.where(mask, eidx_v[pl.ds(off, LANES)], 0)
    er = lax.shift_right_logical(ei, 7)
    ec = lax.bitwise_and(ei, 127)
    vals = plsc.load_gather(y_v, [ni])
    plsc.addupdate_scatter(acc_v, [er, ec], vals, mask=mask)
    plsc.addupdate_scatter(cnt_v, [er, ec], ones, mask=mask)

    pltpu.sync_copy(acc_v, sums_hbm.at[wid])
    pltpu.sync_copy(cnt_v, cnts_hbm.at[wid])


def _finalize_body(s_ref, c_ref, b_ref, out_ref):
    s = jnp.sum(s_ref[...], axis=0)
    c = jnp.sum(c_ref[...], axis=0)
    z = s / jnp.maximum(c, 1.0) + b_ref[0, 0]
    out_ref[...] = jax.nn.sigmoid(z)


@jax.jit
def kernel(X, node_idx, edge_idx, W, b):
    y2d = pl.pallas_call(
        _matvec_body,
        grid=(5,),
        in_specs=[
            pl.BlockSpec((2000, D), lambda i: (i, 0)),
            pl.BlockSpec((D, 1), lambda i: (0, 0)),
        ],
        out_specs=pl.BlockSpec((2000, 1), lambda i: (i, 0)),
        out_shape=jax.ShapeDtypeStruct((N_NODES, 1), jnp.float32),
    )(X, W.reshape(D, 1))
    y = y2d.reshape(N_NODES)

    mesh = plsc.VectorSubcoreMesh(core_axis_name="c", subcore_axis_name="s")
    sc_seg = pl.kernel(
        _sc_body,
        out_type=(
            jax.ShapeDtypeStruct((NWORK, H_ROWS, H_COLS), jnp.float32),
            jax.ShapeDtypeStruct((NWORK, H_ROWS, H_COLS), jnp.float32),
        ),
        mesh=mesh,
        compiler_params=pltpu.CompilerParams(needs_layout_passes=False),
        scratch_types=[
            pltpu.VMEM((10240,), jnp.float32),
            pltpu.VMEM((PPW_PAD,), jnp.int32),
            pltpu.VMEM((PPW_PAD,), jnp.int32),
            pltpu.VMEM((H_ROWS, H_COLS), jnp.float32),
            pltpu.VMEM((H_ROWS, H_COLS), jnp.float32),
        ],
    )
    sums, cnts = sc_seg(y, node_idx, edge_idx)

    out = pl.pallas_call(
        _finalize_body,
        in_specs=[
            pl.BlockSpec((NWORK, H_ROWS, H_COLS), lambda: (0, 0, 0)),
            pl.BlockSpec((NWORK, H_ROWS, H_COLS), lambda: (0, 0, 0)),
            pl.BlockSpec(memory_space=pltpu.SMEM),
        ],
        out_specs=pl.BlockSpec((H_ROWS, H_COLS), lambda: (0, 0)),
        out_shape=jax.ShapeDtypeStruct((H_ROWS, H_COLS), jnp.float32),
    )(sums, cnts, b.reshape(1, 1))

    return out.reshape(H_PAD)[:N_HEDGES].reshape(N_HEDGES, 1)


# speedup vs baseline: 31.4425x; 1.0756x over previous
import functools

import jax
import jax.numpy as jnp
from jax import lax
from jax.experimental import pallas as pl
from jax.experimental.pallas import tpu as pltpu
from jax.experimental.pallas import tpu_sc as plsc

N_NODES = 10000
N_PAIRS = 160000
N_HEDGES = 20000
D = 256

LANES = 16
NWORK = 32
PPW = N_PAIRS // NWORK
NCHUNK = PPW // LANES
TAIL = PPW - NCHUNK * LANES
PPW_PAD = (NCHUNK + 1) * LANES
H_ROWS = 160
H_COLS = 128
H_PAD = H_ROWS * H_COLS
WIN = 16


def _matvec_body(x_ref, wt_ref, y_ref):
    y_ref[...] = jnp.dot(x_ref[...], wt_ref[...],
                         preferred_element_type=jnp.float32)


def _sc_body(y_hbm, nidx_hbm, eidx_hbm, sums_hbm, cnts_hbm, meta_hbm,
             y_v, nidx_v, eidx_v, acc_v, cnt_v, meta_v, y_sem):
    wid = lax.axis_index("s") * 2 + lax.axis_index("c")
    base = wid * PPW

    y_copy = pltpu.make_async_copy(y_hbm, y_v.at[pl.ds(0, N_NODES)], y_sem)
    y_copy.start()
    pltpu.sync_copy(nidx_hbm.at[pl.ds(base, PPW)], nidx_v.at[pl.ds(0, PPW)])
    pltpu.sync_copy(eidx_hbm.at[pl.ds(base, PPW)], eidx_v.at[pl.ds(0, PPW)])

    zeros = jnp.zeros((LANES,), jnp.float32)
    ones = jnp.ones((LANES,), jnp.float32)

    lo_row = lax.shift_right_logical(jnp.min(eidx_v[pl.ds(0, LANES)]), 7)
    hi_row = lax.shift_right_logical(
        jnp.max(eidx_v[pl.ds(PPW - LANES, LANES)]), 7)

    def zero_body(r, carry):
        for c in range(H_COLS // LANES):
            acc_v[r, pl.ds(c * LANES, LANES)] = zeros
            cnt_v[r, pl.ds(c * LANES, LANES)] = zeros
        return carry

    lax.fori_loop(lo_row, hi_row + 1, zero_body, 0)

    def do_chunk(off):
        ni = nidx_v[pl.ds(off, LANES)]
        ei = eidx_v[pl.ds(off, LANES)]
        er = lax.shift_right_logical(ei, 7)
        ec = lax.bitwise_and(ei, 127)
        vals = plsc.load_gather(y_v, [ni])
        plsc.addupdate_scatter(acc_v, [er, ec], vals)
        plsc.addupdate_scatter(cnt_v, [er, ec], ones)

    y_copy.wait()

    UNROLL = 4

    def pair_body(i, carry):
        for u in range(UNROLL):
            do_chunk(i * (LANES * UNROLL) + u * LANES)
        return carry

    lax.fori_loop(0, NCHUNK // UNROLL, pair_body, 0)

    mask = lax.iota(jnp.int32, LANES) < TAIL
    off = NCHUNK * LANES
    ni = jnp.where(mask, nidx_v[pl.ds(off, LANES)], 0)
    ei = jnp.where(mask, eidx_v[pl.ds(off, LANES)], 0)
    er = lax.shift_right_logical(ei, 7)
    ec = lax.bitwise_and(ei, 127)
    vals = plsc.load_gather(y_v, [ni])
    plsc.addupdate_scatter(acc_v, [er, ec], vals, mask=mask)
    plsc.addupdate_scatter(cnt_v, [er, ec], ones, mask=mask)

    start = pl.multiple_of(
        jnp.minimum(lax.bitwise_and(lo_row, ~7), H_ROWS - WIN), 8)
    nrows = hi_row + 1 - start

    def narrow_wb(_):
        pltpu.sync_copy(acc_v.at[pl.ds(start, WIN), :],
                        sums_hbm.at[wid, pl.ds(start, WIN), :])
        pltpu.sync_copy(cnt_v.at[pl.ds(start, WIN), :],
                        cnts_hbm.at[wid, pl.ds(start, WIN), :])
        return 0

    def full_wb(_):
        pltpu.sync_copy(acc_v, sums_hbm.at[wid])
        pltpu.sync_copy(cnt_v, cnts_hbm.at[wid])
        return 0

    lax.cond(nrows <= WIN, narrow_wb, full_wb, 0)

    lanes = lax.iota(jnp.int32, LANES)
    meta = jnp.where(lanes == 0, lo_row, jnp.where(lanes == 1, hi_row, 0))
    meta_v[...] = meta
    pltpu.sync_copy(meta_v, meta_hbm.at[wid])


def _finalize_body(s_ref, c_ref, meta_ref, b_ref, out_ref):
    rows = lax.broadcasted_iota(jnp.int32, (H_ROWS, H_COLS), 0)
    s = jnp.zeros((H_ROWS, H_COLS), jnp.float32)
    c = jnp.zeros((H_ROWS, H_COLS), jnp.float32)
    for w in range(NWORK):
        m = (rows >= meta_ref[w, 0]) & (rows <= meta_ref[w, 1])
        s = s + jnp.where(m, s_ref[w], 0.0)
        c = c + jnp.where(m, c_ref[w], 0.0)
    z = s / jnp.maximum(c, 1.0) + b_ref[0, 0]
    out_ref[...] = jax.nn.sigmoid(z)


@jax.jit
def kernel(X, node_idx, edge_idx, W, b):
    y2d = pl.pallas_call(
        _matvec_body,
        grid=(5,),
        in_specs=[
            pl.BlockSpec((2000, D), lambda i: (i, 0)),
            pl.BlockSpec((D, 1), lambda i: (0, 0)),
        ],
        out_specs=pl.BlockSpec((2000, 1), lambda i: (i, 0)),
        out_shape=jax.ShapeDtypeStruct((N_NODES, 1), jnp.float32),
    )(X, W.reshape(D, 1))
    y = y2d.reshape(N_NODES)

    mesh = plsc.VectorSubcoreMesh(core_axis_name="c", subcore_axis_name="s")
    sc_seg = pl.kernel(
        _sc_body,
        out_type=(
            jax.ShapeDtypeStruct((NWORK, H_ROWS, H_COLS), jnp.float32),
            jax.ShapeDtypeStruct((NWORK, H_ROWS, H_COLS), jnp.float32),
            jax.ShapeDtypeStruct((NWORK, LANES), jnp.int32),
        ),
        mesh=mesh,
        compiler_params=pltpu.CompilerParams(needs_layout_passes=False),
        scratch_types=[
            pltpu.VMEM((10240,), jnp.float32),
            pltpu.VMEM((PPW_PAD,), jnp.int32),
            pltpu.VMEM((PPW_PAD,), jnp.int32),
            pltpu.VMEM((H_ROWS, H_COLS), jnp.float32),
            pltpu.VMEM((H_ROWS, H_COLS), jnp.float32),
            pltpu.VMEM((LANES,), jnp.int32),
            pltpu.SemaphoreType.DMA,
        ],
    )
    sums, cnts, meta = sc_seg(y, node_idx, edge_idx)

    out = pl.pallas_call(
        _finalize_body,
        in_specs=[
            pl.BlockSpec((NWORK, H_ROWS, H_COLS), lambda: (0, 0, 0)),
            pl.BlockSpec((NWORK, H_ROWS, H_COLS), lambda: (0, 0, 0)),
            pl.BlockSpec(memory_space=pltpu.SMEM),
            pl.BlockSpec(memory_space=pltpu.SMEM),
        ],
        out_specs=pl.BlockSpec((H_ROWS, H_COLS), lambda: (0, 0)),
        out_shape=jax.ShapeDtypeStruct((H_ROWS, H_COLS), jnp.float32),
    )(sums, cnts, meta, b.reshape(1, 1))

    return out.reshape(H_PAD)[:N_HEDGES].reshape(N_HEDGES, 1)


# speedup vs baseline: 34.2635x; 1.0897x over previous
import functools

import jax
import jax.numpy as jnp
from jax import lax
from jax.experimental import pallas as pl
from jax.experimental.pallas import tpu as pltpu
from jax.experimental.pallas import tpu_sc as plsc

N_NODES = 10000
N_PAIRS = 160000
N_HEDGES = 20000
D = 256

LANES = 16
NWORK = 32
PPW = N_PAIRS // NWORK
NCHUNK = PPW // LANES
TAIL = PPW - NCHUNK * LANES
PPW_PAD = (NCHUNK + 1) * LANES
H_ROWS = 160
H_COLS = 128
H_PAD = H_ROWS * H_COLS
WIN = 16


def _matvec_body(x_ref, wt_ref, y_ref):
    ycol = jnp.dot(x_ref[...], wt_ref[...],
                   preferred_element_type=jnp.float32)
    y_ref[...] = ycol.reshape(8, 128)


def _sc_body(y_hbm, nidx_hbm, eidx_hbm, sums_hbm, cnts_hbm, meta_hbm,
             y_v, nidx_v, eidx_v, acc_v, cnt_v, meta_v, y_sem):
    wid = lax.axis_index("s") * 2 + lax.axis_index("c")
    base = wid * PPW

    y_copy = pltpu.make_async_copy(y_hbm, y_v, y_sem)
    y_copy.start()
    pltpu.sync_copy(nidx_hbm.at[pl.ds(base, PPW)], nidx_v.at[pl.ds(0, PPW)])
    pltpu.sync_copy(eidx_hbm.at[pl.ds(base, PPW)], eidx_v.at[pl.ds(0, PPW)])

    zeros = jnp.zeros((LANES,), jnp.float32)
    ones = jnp.ones((LANES,), jnp.float32)

    lo_row = lax.shift_right_logical(jnp.min(eidx_v[pl.ds(0, LANES)]), 7)
    hi_row = lax.shift_right_logical(
        jnp.max(eidx_v[pl.ds(PPW - LANES, LANES)]), 7)

    def zero_body(r, carry):
        for c in range(H_COLS // LANES):
            acc_v[r, pl.ds(c * LANES, LANES)] = zeros
            cnt_v[r, pl.ds(c * LANES, LANES)] = zeros
        return carry

    lax.fori_loop(lo_row, hi_row + 1, zero_body, 0)

    def do_chunk(off):
        ni = nidx_v[pl.ds(off, LANES)]
        ei = eidx_v[pl.ds(off, LANES)]
        er = lax.shift_right_logical(ei, 7)
        ec = lax.bitwise_and(ei, 127)
        vals = plsc.load_gather(
            y_v, [lax.shift_right_logical(ni, 7), lax.bitwise_and(ni, 127)])
        plsc.addupdate_scatter(acc_v, [er, ec], vals)
        plsc.addupdate_scatter(cnt_v, [er, ec], ones)

    y_copy.wait()

    UNROLL = 4

    def pair_body(i, carry):
        for u in range(UNROLL):
            do_chunk(i * (LANES * UNROLL) + u * LANES)
        return carry

    lax.fori_loop(0, NCHUNK // UNROLL, pair_body, 0)

    mask = lax.iota(jnp.int32, LANES) < TAIL
    off = NCHUNK * LANES
    ni = jnp.where(mask, nidx_v[pl.ds(off, LANES)], 0)
    ei = jnp.where(mask, eidx_v[pl.ds(off, LANES)], 0)
    er = lax.shift_right_logical(ei, 7)
    ec = lax.bitwise_and(ei, 127)
    vals = plsc.load_gather(
        y_v, [lax.shift_right_logical(ni, 7), lax.bitwise_and(ni, 127)])
    plsc.addupdate_scatter(acc_v, [er, ec], vals, mask=mask)
    plsc.addupdate_scatter(cnt_v, [er, ec], ones, mask=mask)

    start = pl.multiple_of(
        jnp.minimum(lax.bitwise_and(lo_row, ~7), H_ROWS - WIN), 8)
    nrows = hi_row + 1 - start

    def narrow_wb(_):
        pltpu.sync_copy(acc_v.at[pl.ds(start, WIN), :],
                        sums_hbm.at[wid, pl.ds(start, WIN), :])
        pltpu.sync_copy(cnt_v.at[pl.ds(start, WIN), :],
                        cnts_hbm.at[wid, pl.ds(start, WIN), :])
        return 0

    def full_wb(_):
        pltpu.sync_copy(acc_v, sums_hbm.at[wid])
        pltpu.sync_copy(cnt_v, cnts_hbm.at[wid])
        return 0

    lax.cond(nrows <= WIN, narrow_wb, full_wb, 0)

    lanes = lax.iota(jnp.int32, LANES)
    meta = jnp.where(lanes == 0, lo_row, jnp.where(lanes == 1, hi_row, 0))
    meta_v[...] = meta
    pltpu.sync_copy(meta_v, meta_hbm.at[wid])


def _finalize_body(s_ref, c_ref, meta_ref, b_ref, out_ref):
    rows = lax.broadcasted_iota(jnp.int32, (H_ROWS, H_COLS), 0)
    s = jnp.zeros((H_ROWS, H_COLS), jnp.float32)
    c = jnp.zeros((H_ROWS, H_COLS), jnp.float32)
    for w in range(NWORK):
        m = (rows >= meta_ref[w, 0]) & (rows <= meta_ref[w, 1])
        s = s + jnp.where(m, s_ref[w], 0.0)
        c = c + jnp.where(m, c_ref[w], 0.0)
    z = s / jnp.maximum(c, 1.0) + b_ref[0, 0]
    out_ref[...] = jax.nn.sigmoid(z)


@jax.jit
def kernel(X, node_idx, edge_idx, W, b):
    y = pl.pallas_call(
        _matvec_body,
        grid=(10,),
        in_specs=[
            pl.BlockSpec((1024, D), lambda i: (i, 0)),
            pl.BlockSpec((D, 1), lambda i: (0, 0)),
        ],
        out_specs=pl.BlockSpec((8, 128), lambda i: (i, 0)),
        out_shape=jax.ShapeDtypeStruct((80, 128), jnp.float32),
    )(X, W.reshape(D, 1))

    mesh = plsc.VectorSubcoreMesh(core_axis_name="c", subcore_axis_name="s")
    sc_seg = pl.kernel(
        _sc_body,
        out_type=(
            jax.ShapeDtypeStruct((NWORK, H_ROWS, H_COLS), jnp.float32),
            jax.ShapeDtypeStruct((NWORK, H_ROWS, H_COLS), jnp.float32),
            jax.ShapeDtypeStruct((NWORK, LANES), jnp.int32),
        ),
        mesh=mesh,
        compiler_params=pltpu.CompilerParams(needs_layout_passes=False),
        scratch_types=[
            pltpu.VMEM((80, 128), jnp.float32),
            pltpu.VMEM((PPW_PAD,), jnp.int32),
            pltpu.VMEM((PPW_PAD,), jnp.int32),
            pltpu.VMEM((H_ROWS, H_COLS), jnp.float32),
            pltpu.VMEM((H_ROWS, H_COLS), jnp.float32),
            pltpu.VMEM((LANES,), jnp.int32),
            pltpu.SemaphoreType.DMA,
        ],
    )
    sums, cnts, meta = sc_seg(y, node_idx, edge_idx)

    out = pl.pallas_call(
        _finalize_body,
        in_specs=[
            pl.BlockSpec((NWORK, H_ROWS, H_COLS), lambda: (0, 0, 0)),
            pl.BlockSpec((NWORK, H_ROWS, H_COLS), lambda: (0, 0, 0)),
            pl.BlockSpec(memory_space=pltpu.SMEM),
            pl.BlockSpec(memory_space=pltpu.SMEM),
        ],
        out_specs=pl.BlockSpec((H_ROWS, H_COLS), lambda: (0, 0)),
        out_shape=jax.ShapeDtypeStruct((H_ROWS, H_COLS), jnp.float32),
    )(sums, cnts, meta, b.reshape(1, 1))

    return out.reshape(H_PAD)[:N_HEDGES].reshape(N_HEDGES, 1)
